# baseline (speedup 1.0000x reference)
import jax
import jax.numpy as jnp
from jax.experimental import pallas as pl
from jax.experimental.pallas import tpu as pltpu

_B, _S, _D, _E = 2, 2048, 1024, 8
_TS = 1024
_EG = 2


def _router_kernel(h_ref, qw_ref, cw_ref, out_ref, q_scr, qn2_scr):
    s = pl.program_id(1)
    eg = pl.program_id(2)

    x = h_ref[0]

    @pl.when(eg == 0)
    def _compute_query():
        q0 = jax.lax.dot_general(
            x, qw_ref[...], (((1,), (1,)), ((), ())),
            preferred_element_type=jnp.float32)
        q_scr[...] = q0
        qn2_scr[...] = jnp.sum(q0 * q0, axis=1, keepdims=True)

    q = q_scr[...]
    qn2 = qn2_scr[...]

    lane = jax.lax.broadcasted_iota(jnp.int32, (1, _E), 1)
    acc = jnp.zeros((1, _E), dtype=jnp.float32)
    for j in range(_EG):
        k = jax.lax.dot_general(
            x, cw_ref[j], (((1,), (1,)), ((), ())),
            preferred_element_type=jnp.float32)
        num = jnp.sum(q * k, axis=1, keepdims=True)
        kn2 = jnp.sum(k * k, axis=1, keepdims=True)
        denom = jnp.maximum(jnp.sqrt(qn2 * kn2), 1e-8)
        part = jnp.sum(num / denom) * (1.0 / _S)
        acc = acc + jnp.where(lane == eg * _EG + j, part, 0.0)

    @pl.when((s == 0) & (eg == 0))
    def _init():
        out_ref[...] = jnp.zeros_like(out_ref)

    out_ref[...] += acc[None]


def kernel(h, q_W, chip_weights):
    n_s_tiles = _S // _TS
    out = pl.pallas_call(
        _router_kernel,
        grid=(_B, n_s_tiles, _E // _EG),
        in_specs=[
            pl.BlockSpec((1, _TS, _D), lambda b, s, eg: (b, s, 0)),
            pl.BlockSpec((_D, _D), lambda b, s, eg: (0, 0)),
            pl.BlockSpec((_EG, _D, _D), lambda b, s, eg: (eg, 0, 0)),
        ],
        out_specs=pl.BlockSpec((1, 1, _E), lambda b, s, eg: (b, 0, 0)),
        out_shape=jax.ShapeDtypeStruct((_B, 1, _E), jnp.float32),
        scratch_shapes=[
            pltpu.VMEM((_TS, _D), jnp.float32),
            pltpu.VMEM((_TS, 1), jnp.float32),
        ],
        compiler_params=pltpu.CompilerParams(
            dimension_semantics=("arbitrary", "arbitrary", "arbitrary"),
        ),
    )(h, q_W, chip_weights)
    return out.reshape(_B, _E)
